# baseline (speedup 1.0000x reference)
import functools

import jax
import jax.numpy as jnp
from jax import lax
from jax.experimental import pallas as pl
from jax.experimental.pallas import tpu as pltpu
from jax.experimental.pallas import tpu_sc as plsc

N = 10000
E = 160000
D_IN = 256
D_HID = 128
D_OUT = 128
DH = 64

NC = 2
NS = 16
NW = NC * NS
CHUNK = 128
NCHUNK = 40
E_PAD = NW * NCHUNK * CHUNK
NCHUNK2 = E_PAD // (NS * CHUNK)
N_PAD = 10240
RPT = N_PAD // NS


def _scatter_body(g_hbm, src_hbm, dst_hbm, zeros_hbm, out_hbm, sidx_v, didx_v,
                  rows_v, acc_sh, sem):
  cid = lax.axis_index("c")
  sid = lax.axis_index("s")
  wid = cid * NS + sid
  pltpu.sync_copy(zeros_hbm, acc_sh.at[pl.ds(sid * RPT, RPT)])
  pltpu.sync_copy(src_hbm.at[wid], sidx_v)
  pltpu.sync_copy(dst_hbm.at[wid], didx_v)
  plsc.subcore_barrier()

  def body(j, carry):
    pltpu.async_copy(g_hbm.at[sidx_v.at[j]], rows_v, sem).wait()
    pltpu.sync_copy(rows_v, acc_sh.at[didx_v.at[j]], add=True)
    return carry

  lax.fori_loop(0, NCHUNK, body, 0)
  plsc.subcore_barrier()
  pltpu.sync_copy(acc_sh.at[pl.ds(sid * RPT, RPT)],
                  out_hbm.at[pl.ds(cid * N_PAD + sid * RPT, RPT)])


@functools.cache
def _scatter_sc():
  mesh = plsc.VectorSubcoreMesh(
      core_axis_name="c", subcore_axis_name="s", num_cores=NC,
      num_subcores=NS)
  return pl.kernel(
      _scatter_body,
      out_type=jax.ShapeDtypeStruct((NC * N_PAD, D_HID), jnp.float32),
      mesh=mesh,
      scratch_types=[
          pltpu.VMEM((NCHUNK, CHUNK), jnp.int32),
          pltpu.VMEM((NCHUNK, CHUNK), jnp.int32),
          pltpu.VMEM((CHUNK, D_HID), jnp.float32),
          pltpu.VMEM_SHARED((N_PAD, D_HID), jnp.float32),
          pltpu.SemaphoreType.DMA,
      ],
  )


_BR = 400


def _dis_from(deg_ref):
  deg = deg_ref[0] + deg_ref[1] + 1.0
  return lax.rsqrt(deg[:, :1])


def _mm1_body(x_ref, w_ref, deg_ref, out_ref):
  dis = _dis_from(deg_ref)
  hw = jnp.dot(x_ref[...], w_ref[...], preferred_element_type=jnp.float32)
  out_ref[...] = hw * dis


def _mm2_body(acc_ref, g_ref, deg_ref, b_ref, w_ref, out_ref):
  dis = _dis_from(deg_ref)
  h = dis * (acc_ref[0] + acc_ref[1] + g_ref[...]) + b_ref[...]
  h = jnp.maximum(h, 0.0)
  hw = jnp.dot(h, w_ref[...], preferred_element_type=jnp.float32)
  out_ref[...] = hw * dis


def _fin_body(acc_ref, g_ref, deg_ref, b_ref, out_ref):
  dis = _dis_from(deg_ref)
  out_ref[...] = dis * (acc_ref[0] + acc_ref[1] + g_ref[...]) + b_ref[...]


_ACC_SPEC = pl.BlockSpec((2, _BR, D_HID), lambda i: (0, i, 0))
_G_SPEC = pl.BlockSpec((_BR, D_HID), lambda i: (i, 0))
_DEG_SPEC = pl.BlockSpec((2, _BR, 16), lambda i: (0, i, 0))


def _mm1(x, w1, deg2):
  return pl.pallas_call(
      _mm1_body,
      grid=(N // _BR,),
      in_specs=[
          pl.BlockSpec((_BR, D_IN), lambda i: (i, 0)),
          pl.BlockSpec((D_IN, D_HID), lambda i: (0, 0)),
          _DEG_SPEC,
      ],
      out_specs=pl.BlockSpec((_BR, D_HID), lambda i: (i, 0)),
      out_shape=jax.ShapeDtypeStruct((N, D_HID), jnp.float32),
  )(x, w1, deg2)


def _mm2(acc2, g, deg2, b1, w2):
  return pl.pallas_call(
      _mm2_body,
      grid=(N // _BR,),
      in_specs=[
          _ACC_SPEC,
          _G_SPEC,
          _DEG_SPEC,
          pl.BlockSpec((1, D_HID), lambda i: (0, 0)),
          pl.BlockSpec((D_HID, D_OUT), lambda i: (0, 0)),
      ],
      out_specs=pl.BlockSpec((_BR, D_OUT), lambda i: (i, 0)),
      out_shape=jax.ShapeDtypeStruct((N, D_OUT), jnp.float32),
  )(acc2, g, deg2, b1, w2)


def _fin(acc2, g, deg2, b2):
  return pl.pallas_call(
      _fin_body,
      grid=(N // _BR,),
      in_specs=[
          _ACC_SPEC,
          _G_SPEC,
          _DEG_SPEC,
          pl.BlockSpec((1, D_OUT), lambda i: (0, 0)),
      ],
      out_specs=pl.BlockSpec((_BR, D_OUT), lambda i: (i, 0)),
      out_shape=jax.ShapeDtypeStruct((N, D_OUT), jnp.float32),
  )(acc2, g, deg2, b2)


@jax.jit
def _run(x, edge_index, w1, b1, w2, b2):
  ei = edge_index.astype(jnp.int32)
  pad = E_PAD - E
  src_f = jnp.concatenate([ei[0], jnp.zeros((pad,), jnp.int32)])
  dst_f = jnp.concatenate([ei[1], jnp.full((pad,), N, jnp.int32)])
  src32 = src_f.reshape(NW, NCHUNK, CHUNK)
  dst32 = dst_f.reshape(NW, NCHUNK, CHUNK)

  zeros128 = jnp.zeros((RPT, D_HID), jnp.float32)

  deg = jax.ops.segment_sum(jnp.ones((E,), jnp.float32), ei[1],
                            num_segments=N)
  deg2 = jnp.zeros((NC, N_PAD, 16), jnp.float32).at[0, :N, :].set(
      deg[:, None])

  g1 = _mm1(x, w1, deg2)
  acc1 = _scatter_sc()(g1, src32, dst32, zeros128).reshape(NC, N_PAD, D_HID)
  g2 = _mm2(acc1, g1, deg2, b1.reshape(1, D_HID), w2)
  acc2 = _scatter_sc()(g2, src32, dst32, zeros128).reshape(NC, N_PAD, D_OUT)
  return _fin(acc2, g2, deg2, b2.reshape(1, D_OUT))


def kernel(x, edge_index, cache_name, W1, b1, W2, b2):
  return _run(x, edge_index, W1, b1, W2, b2)


# speedup vs baseline: 1.0116x; 1.0116x over previous
import functools

import jax
import jax.numpy as jnp
from jax import lax
from jax.experimental import pallas as pl
from jax.experimental.pallas import tpu as pltpu
from jax.experimental.pallas import tpu_sc as plsc

N = 10000
E = 160000
D_IN = 256
D_HID = 128
D_OUT = 128
DH = 64

NC = 2
NS = 16
NW = NC * NS
CHUNK = 128
NCHUNK = 40
E_PAD = NW * NCHUNK * CHUNK
NCHUNK2 = E_PAD // (NS * CHUNK)
N_PAD = 10240
RPT = N_PAD // NS


def _scatter_body(g_hbm, src_hbm, dst_hbm, zeros_hbm, out_hbm, sidx_v, didx_v,
                  rows_v, acc_sh, sem):
  cid = lax.axis_index("c")
  sid = lax.axis_index("s")
  wid = cid * NS + sid
  pltpu.sync_copy(zeros_hbm, acc_sh.at[pl.ds(sid * RPT, RPT)])
  pltpu.sync_copy(src_hbm.at[wid], sidx_v)
  pltpu.sync_copy(dst_hbm.at[wid], didx_v)
  plsc.subcore_barrier()

  @pl.loop(0, NCHUNK, step=2)
  def _(j):
    cp0 = pltpu.async_copy(g_hbm.at[sidx_v.at[j]], rows_v.at[0], sem)
    cp1 = pltpu.async_copy(g_hbm.at[sidx_v.at[j + 1]], rows_v.at[1], sem)
    cp0.wait()
    cp1.wait()
    pltpu.sync_copy(rows_v.at[0], acc_sh.at[didx_v.at[j]], add=True)
    pltpu.sync_copy(rows_v.at[1], acc_sh.at[didx_v.at[j + 1]], add=True)
  plsc.subcore_barrier()
  pltpu.sync_copy(acc_sh.at[pl.ds(sid * RPT, RPT)],
                  out_hbm.at[pl.ds(cid * N_PAD + sid * RPT, RPT)])


@functools.cache
def _scatter_sc():
  mesh = plsc.VectorSubcoreMesh(
      core_axis_name="c", subcore_axis_name="s", num_cores=NC,
      num_subcores=NS)
  return pl.kernel(
      _scatter_body,
      out_type=jax.ShapeDtypeStruct((NC * N_PAD, D_HID), jnp.float32),
      mesh=mesh,
      scratch_types=[
          pltpu.VMEM((NCHUNK, CHUNK), jnp.int32),
          pltpu.VMEM((NCHUNK, CHUNK), jnp.int32),
          pltpu.VMEM((2, CHUNK, D_HID), jnp.float32),
          pltpu.VMEM_SHARED((N_PAD, D_HID), jnp.float32),
          pltpu.SemaphoreType.DMA,
      ],
  )


_BR = 400


def _dis_from(deg_ref):
  deg = deg_ref[0] + deg_ref[1] + 1.0
  return lax.rsqrt(deg[:, :1])


def _mm1_body(x_ref, w_ref, deg_ref, out_ref):
  dis = _dis_from(deg_ref)
  hw = jnp.dot(x_ref[...], w_ref[...], preferred_element_type=jnp.float32)
  out_ref[...] = hw * dis


def _mm2_body(acc_ref, g_ref, deg_ref, b_ref, w_ref, out_ref):
  dis = _dis_from(deg_ref)
  h = dis * (acc_ref[0] + acc_ref[1] + g_ref[...]) + b_ref[...]
  h = jnp.maximum(h, 0.0)
  hw = jnp.dot(h, w_ref[...], preferred_element_type=jnp.float32)
  out_ref[...] = hw * dis


def _fin_body(acc_ref, g_ref, deg_ref, b_ref, out_ref):
  dis = _dis_from(deg_ref)
  out_ref[...] = dis * (acc_ref[0] + acc_ref[1] + g_ref[...]) + b_ref[...]


_ACC_SPEC = pl.BlockSpec((2, _BR, D_HID), lambda i: (0, i, 0))
_G_SPEC = pl.BlockSpec((_BR, D_HID), lambda i: (i, 0))
_DEG_SPEC = pl.BlockSpec((2, _BR, 16), lambda i: (0, i, 0))


def _mm1(x, w1, deg2):
  return pl.pallas_call(
      _mm1_body,
      grid=(N // _BR,),
      in_specs=[
          pl.BlockSpec((_BR, D_IN), lambda i: (i, 0)),
          pl.BlockSpec((D_IN, D_HID), lambda i: (0, 0)),
          _DEG_SPEC,
      ],
      out_specs=pl.BlockSpec((_BR, D_HID), lambda i: (i, 0)),
      out_shape=jax.ShapeDtypeStruct((N, D_HID), jnp.float32),
  )(x, w1, deg2)


def _mm2(acc2, g, deg2, b1, w2):
  return pl.pallas_call(
      _mm2_body,
      grid=(N // _BR,),
      in_specs=[
          _ACC_SPEC,
          _G_SPEC,
          _DEG_SPEC,
          pl.BlockSpec((1, D_HID), lambda i: (0, 0)),
          pl.BlockSpec((D_HID, D_OUT), lambda i: (0, 0)),
      ],
      out_specs=pl.BlockSpec((_BR, D_OUT), lambda i: (i, 0)),
      out_shape=jax.ShapeDtypeStruct((N, D_OUT), jnp.float32),
  )(acc2, g, deg2, b1, w2)


def _fin(acc2, g, deg2, b2):
  return pl.pallas_call(
      _fin_body,
      grid=(N // _BR,),
      in_specs=[
          _ACC_SPEC,
          _G_SPEC,
          _DEG_SPEC,
          pl.BlockSpec((1, D_OUT), lambda i: (0, 0)),
      ],
      out_specs=pl.BlockSpec((_BR, D_OUT), lambda i: (i, 0)),
      out_shape=jax.ShapeDtypeStruct((N, D_OUT), jnp.float32),
  )(acc2, g, deg2, b2)


@jax.jit
def _run(x, edge_index, w1, b1, w2, b2):
  ei = edge_index.astype(jnp.int32)
  pad = E_PAD - E
  src_f = jnp.concatenate([ei[0], jnp.zeros((pad,), jnp.int32)])
  dst_f = jnp.concatenate([ei[1], jnp.full((pad,), N, jnp.int32)])
  src32 = src_f.reshape(NW, NCHUNK, CHUNK)
  dst32 = dst_f.reshape(NW, NCHUNK, CHUNK)

  zeros128 = jnp.zeros((RPT, D_HID), jnp.float32)

  deg = jax.ops.segment_sum(jnp.ones((E,), jnp.float32), ei[1],
                            num_segments=N)
  deg2 = jnp.zeros((NC, N_PAD, 16), jnp.float32).at[0, :N, :].set(
      deg[:, None])

  g1 = _mm1(x, w1, deg2)
  acc1 = _scatter_sc()(g1, src32, dst32, zeros128).reshape(NC, N_PAD, D_HID)
  g2 = _mm2(acc1, g1, deg2, b1.reshape(1, D_HID), w2)
  acc2 = _scatter_sc()(g2, src32, dst32, zeros128).reshape(NC, N_PAD, D_OUT)
  return _fin(acc2, g2, deg2, b2.reshape(1, D_OUT))


def kernel(x, edge_index, cache_name, W1, b1, W2, b2):
  return _run(x, edge_index, W1, b1, W2, b2)


# speedup vs baseline: 1.0124x; 1.0007x over previous
import functools

import jax
import jax.numpy as jnp
from jax import lax
from jax.experimental import pallas as pl
from jax.experimental.pallas import tpu as pltpu
from jax.experimental.pallas import tpu_sc as plsc

N = 10000
E = 160000
D_IN = 256
D_HID = 128
D_OUT = 128

NC = 2
NS = 16
NW = NC * NS
CHUNK = 128
NCHUNK = 40
E_PAD = NW * NCHUNK * CHUNK
N_PAD = 10240
RPT = N_PAD // NS


def _scatter_body(g_hbm, src_hbm, dst_hbm, zeros_hbm, out_hbm, sidx_v, didx_v,
                  rows_v, acc_sh, sem):
  cid = lax.axis_index("c")
  sid = lax.axis_index("s")
  wid = cid * NS + sid
  pltpu.sync_copy(zeros_hbm, acc_sh.at[pl.ds(sid * RPT, RPT)])
  pltpu.sync_copy(src_hbm.at[wid], sidx_v)
  pltpu.sync_copy(dst_hbm.at[wid], didx_v)
  plsc.subcore_barrier()

  @pl.loop(0, NCHUNK, step=2)
  def _(j):
    cp0 = pltpu.async_copy(g_hbm.at[sidx_v.at[j]], rows_v.at[0], sem)
    cp1 = pltpu.async_copy(g_hbm.at[sidx_v.at[j + 1]], rows_v.at[1], sem)
    cp0.wait()
    cp1.wait()
    pltpu.sync_copy(rows_v.at[0], acc_sh.at[didx_v.at[j]], add=True)
    pltpu.sync_copy(rows_v.at[1], acc_sh.at[didx_v.at[j + 1]], add=True)
  plsc.subcore_barrier()
  pltpu.sync_copy(acc_sh.at[pl.ds(sid * RPT, RPT)],
                  out_hbm.at[pl.ds(cid * N_PAD + sid * RPT, RPT)])


@functools.cache
def _scatter_sc():
  mesh = plsc.VectorSubcoreMesh(
      core_axis_name="c", subcore_axis_name="s", num_cores=NC,
      num_subcores=NS)
  return pl.kernel(
      _scatter_body,
      out_type=jax.ShapeDtypeStruct((NC * N_PAD, D_HID), jnp.float32),
      mesh=mesh,
      scratch_types=[
          pltpu.VMEM((NCHUNK, CHUNK), jnp.int32),
          pltpu.VMEM((NCHUNK, CHUNK), jnp.int32),
          pltpu.VMEM((2, CHUNK, D_HID), jnp.float32),
          pltpu.VMEM_SHARED((N_PAD, D_HID), jnp.float32),
          pltpu.SemaphoreType.DMA,
      ],
  )


_BR = 400


def _dis_from(deg_ref):
  deg = deg_ref[0] + deg_ref[1] + 1.0
  return lax.rsqrt(deg[:, :1])


def _mm1_body(x_ref, w_ref, deg_ref, out_ref):
  dis = _dis_from(deg_ref)
  hw = jnp.dot(x_ref[...], w_ref[...], preferred_element_type=jnp.float32)
  out_ref[...] = hw * dis


def _mm2_body(acc_ref, g_ref, deg_ref, b_ref, w_ref, out_ref):
  dis = _dis_from(deg_ref)
  h = dis * (acc_ref[0] + acc_ref[1] + g_ref[...]) + b_ref[...]
  h = jnp.maximum(h, 0.0)
  hw = jnp.dot(h, w_ref[...], preferred_element_type=jnp.float32)
  out_ref[...] = hw * dis


def _fin_body(acc_ref, g_ref, deg_ref, b_ref, out_ref):
  dis = _dis_from(deg_ref)
  out_ref[...] = dis * (acc_ref[0] + acc_ref[1] + g_ref[...]) + b_ref[...]


_ACC_SPEC = pl.BlockSpec((2, _BR, D_HID), lambda i: (0, i, 0))
_G_SPEC = pl.BlockSpec((_BR, D_HID), lambda i: (i, 0))
_DEG_SPEC = pl.BlockSpec((2, _BR, 16), lambda i: (0, i, 0))


def _mm1(x, w1, deg2):
  return pl.pallas_call(
      _mm1_body,
      grid=(N // _BR,),
      in_specs=[
          pl.BlockSpec((_BR, D_IN), lambda i: (i, 0)),
          pl.BlockSpec((D_IN, D_HID), lambda i: (0, 0)),
          _DEG_SPEC,
      ],
      out_specs=pl.BlockSpec((_BR, D_HID), lambda i: (i, 0)),
      out_shape=jax.ShapeDtypeStruct((N, D_HID), jnp.float32),
  )(x, w1, deg2)


def _mm2(acc2, g, deg2, b1, w2):
  return pl.pallas_call(
      _mm2_body,
      grid=(N // _BR,),
      in_specs=[
          _ACC_SPEC,
          _G_SPEC,
          _DEG_SPEC,
          pl.BlockSpec((1, D_HID), lambda i: (0, 0)),
          pl.BlockSpec((D_HID, D_OUT), lambda i: (0, 0)),
      ],
      out_specs=pl.BlockSpec((_BR, D_OUT), lambda i: (i, 0)),
      out_shape=jax.ShapeDtypeStruct((N, D_OUT), jnp.float32),
  )(acc2, g, deg2, b1, w2)


def _fin(acc2, g, deg2, b2):
  return pl.pallas_call(
      _fin_body,
      grid=(N // _BR,),
      in_specs=[
          _ACC_SPEC,
          _G_SPEC,
          _DEG_SPEC,
          pl.BlockSpec((1, D_OUT), lambda i: (0, 0)),
      ],
      out_specs=pl.BlockSpec((_BR, D_OUT), lambda i: (i, 0)),
      out_shape=jax.ShapeDtypeStruct((N, D_OUT), jnp.float32),
  )(acc2, g, deg2, b2)


@jax.jit
def _run(x, edge_index, w1, b1, w2, b2):
  ei = edge_index.astype(jnp.int32)
  pad = E_PAD - E
  src_f = jnp.concatenate([ei[0], jnp.zeros((pad,), jnp.int32)])
  dst_f = jnp.concatenate([ei[1], jnp.full((pad,), N, jnp.int32)])
  src32 = src_f.reshape(NW, NCHUNK, CHUNK)
  dst32 = dst_f.reshape(NW, NCHUNK, CHUNK)

  zeros128 = jnp.zeros((RPT, D_HID), jnp.float32)

  deg = jax.ops.segment_sum(jnp.ones((E,), jnp.float32), ei[1],
                            num_segments=N)
  deg2 = jnp.zeros((NC, N_PAD, 16), jnp.float32).at[0, :N, :].set(
      deg[:, None])

  g1 = _mm1(x, w1, deg2)
  acc1 = _scatter_sc()(g1, src32, dst32, zeros128).reshape(NC, N_PAD, D_HID)
  g2 = _mm2(acc1, g1, deg2, b1.reshape(1, D_HID), w2)
  acc2 = _scatter_sc()(g2, src32, dst32, zeros128).reshape(NC, N_PAD, D_OUT)
  return _fin(acc2, g2, deg2, b2.reshape(1, D_OUT))


def kernel(x, edge_index, cache_name, W1, b1, W2, b2):
  return _run(x, edge_index, W1, b1, W2, b2)
